# baseline (speedup 1.0000x reference)
import functools

import jax
import jax.numpy as jnp
from jax import lax
from jax.experimental import pallas as pl
from jax.experimental.pallas import tpu as pltpu
from jax.experimental.pallas import tpu_sc as plsc

N_EMBD = 64
POS_ROWS = 32
NW = 32
CH = 128
LANES = 16


def _chunk_add_pos(rows_v, pos_v):
    for cb in range(N_EMBD // LANES):
        sl = pl.ds(cb * LANES, LANES)
        pos_regs = [pos_v[r, sl] for r in range(POS_ROWS)]
        for j in range(CH):
            rows_v[j, sl] = rows_v[j, sl] + pos_regs[j % POS_ROWS]


def _make_gather(n_rows):
    g_per_w = n_rows // (NW * CH)
    rw = g_per_w * CH
    mesh = plsc.VectorSubcoreMesh(core_axis_name="c", subcore_axis_name="s")

    @functools.partial(
        pl.kernel,
        mesh=mesh,
        compiler_params=pltpu.CompilerParams(use_tc_tiling_on_sc=False),
        out_type=jax.ShapeDtypeStruct((n_rows, N_EMBD), jnp.float32),
        scratch_types=[
            pltpu.VMEM((g_per_w, CH), jnp.int32),
            pltpu.VMEM((CH, N_EMBD), jnp.float32),
            pltpu.VMEM((POS_ROWS, N_EMBD), jnp.float32),
            pltpu.SemaphoreType.DMA,
        ],
    )
    def gather_add(x_hbm, tok_hbm, pos_hbm, out_hbm, idx_v, rows_v, pos_v, sem):
        cid = lax.axis_index("c")
        sid = lax.axis_index("s")
        wid = sid * 2 + cid
        pltpu.sync_copy(x_hbm.at[wid], idx_v)
        pltpu.sync_copy(pos_hbm, pos_v)
        base = wid * rw

        def chunk(g, carry):
            pltpu.async_copy(tok_hbm.at[idx_v.at[g]], rows_v, sem).wait()
            _chunk_add_pos(rows_v, pos_v)
            off = pl.multiple_of(base + g * CH, CH)
            pltpu.sync_copy(rows_v, out_hbm.at[pl.ds(off, CH)])
            return carry

        lax.fori_loop(0, g_per_w, chunk, 0)

    return gather_add


def kernel(x, token_table, pos_table):
    B, T, C = x.shape
    n_rows = B * T * C
    xr = x.reshape(NW, n_rows // (NW * CH), CH).astype(jnp.int32)
    out = _make_gather(n_rows)(xr, token_table, pos_table)
    return out.reshape(B, T, C, N_EMBD)


# speedup vs baseline: 1.1486x; 1.1486x over previous
import functools

import jax
import jax.numpy as jnp
from jax import lax
from jax.experimental import pallas as pl
from jax.experimental.pallas import tpu as pltpu
from jax.experimental.pallas import tpu_sc as plsc

N_EMBD = 64
POS_ROWS = 32
NW = 32
CH = 128
K = 4
NBUF = 2
LANES = 16


def _chunk_add_pos(chunk_ref, pos_v):
    for cb in range(N_EMBD // LANES):
        sl = pl.ds(cb * LANES, LANES)
        pos_regs = [pos_v[r, sl] for r in range(POS_ROWS)]
        for j in range(CH):
            chunk_ref[j, sl] = chunk_ref[j, sl] + pos_regs[j % POS_ROWS]


def _make_gather(n_rows):
    n_chunks = n_rows // CH
    g_per_w = n_chunks // NW
    nb = g_per_w // K
    assert nb % NBUF == 0
    mesh = plsc.VectorSubcoreMesh(core_axis_name="c", subcore_axis_name="s")

    @functools.partial(
        pl.kernel,
        mesh=mesh,
        compiler_params=pltpu.CompilerParams(use_tc_tiling_on_sc=False),
        out_type=jax.ShapeDtypeStruct((n_chunks, CH, N_EMBD), jnp.float32),
        scratch_types=[
            pltpu.VMEM((g_per_w, CH), jnp.int32),
            pltpu.VMEM((NBUF, K, CH, N_EMBD), jnp.float32),
            pltpu.VMEM((POS_ROWS, N_EMBD), jnp.float32),
            pltpu.SemaphoreType.DMA,
            pltpu.SemaphoreType.DMA,
            pltpu.SemaphoreType.DMA,
            pltpu.SemaphoreType.DMA,
        ],
    )
    def gather_add(x_hbm, tok_hbm, pos_hbm, out_hbm, idx_v, rows_v, pos_v,
                   gsem0, gsem1, osem0, osem1):
        gsem = [gsem0, gsem1]
        osem = [osem0, osem1]
        cid = lax.axis_index("c")
        sid = lax.axis_index("s")
        wid = sid * 2 + cid
        pltpu.sync_copy(x_hbm.at[wid], idx_v)
        pltpu.sync_copy(pos_hbm, pos_v)
        chunk_base = wid * g_per_w

        def fire_gathers(i, b):
            for j in range(K):
                pltpu.async_copy(
                    tok_hbm.at[idx_v.at[i * K + j]], rows_v.at[b, j], gsem[b])

        def drain_gathers(b):
            for j in range(K):
                pltpu.make_async_copy(
                    tok_hbm.at[pl.ds(0, CH)], rows_v.at[b, j], gsem[b]).wait()

        def drain_out(b):
            pltpu.make_async_copy(
                rows_v.at[b], out_hbm.at[pl.ds(0, K)], osem[b]).wait()

        def process(i, b):
            drain_gathers(b)

            def add_sub(s, carry):
                _chunk_add_pos(rows_v.at[b, s], pos_v)
                return carry

            lax.fori_loop(0, K, add_sub, 0)
            off = pl.multiple_of(chunk_base + i * K, K)
            pltpu.async_copy(rows_v.at[b], out_hbm.at[pl.ds(off, K)], osem[b])

        fire_gathers(0, 0)

        def outer(it, carry):
            i0 = it * NBUF
            process(i0, 0)

            @pl.when(it > 0)
            def _():
                drain_out(1)

            fire_gathers(i0 + 1, 1)
            process(i0 + 1, 1)

            @pl.when(it < nb // NBUF - 1)
            def _():
                drain_out(0)
                fire_gathers(i0 + NBUF, 0)

            return carry

        lax.fori_loop(0, nb // NBUF, outer, 0)
        drain_out(0)
        drain_out(1)

    return gather_add


def kernel(x, token_table, pos_table):
    B, T, C = x.shape
    n_rows = B * T * C
    xr = x.reshape(NW, n_rows // (NW * CH), CH).astype(jnp.int32)
    out = _make_gather(n_rows)(xr, token_table, pos_table)
    return out.reshape(B, T, C, N_EMBD)
